# Initial kernel scaffold; baseline (speedup 1.0000x reference)
import jax
import jax.numpy as jnp
from jax.experimental import pallas as pl


def kernel(x, bins):
    raise NotImplementedError("write your pallas kernel here")


# speedup vs baseline: 1.5443x; 1.5443x over previous
import jax
import jax.numpy as jnp
from jax.experimental import pallas as pl
from jax.experimental.pallas import tpu as pltpu

ALPHA = -50.0


def _body(x_ref, bins_ref, o_ref):
    v = x_ref[...]
    b = [bins_ref[k] for k in range(4)]
    d = [jnp.abs(v - bk) for bk in b]
    m = jnp.minimum(jnp.minimum(d[0], d[1]), jnp.minimum(d[2], d[3]))
    e = [jnp.exp(ALPHA * (dk - m)) for dk in d]
    num = b[0] * e[0] + b[1] * e[1] + b[2] * e[2] + b[3] * e[3]
    den = e[0] + e[1] + e[2] + e[3]
    o_ref[...] = num / den


def kernel(x, bins):
    B, F, C = x.shape
    x2 = x.reshape(B, F * C)
    blk = 128
    grid = (B // blk,)
    out = pl.pallas_call(
        _body,
        grid=grid,
        in_specs=[
            pl.BlockSpec((blk, F * C), lambda i: (i, 0)),
            pl.BlockSpec(memory_space=pltpu.SMEM),
        ],
        out_specs=pl.BlockSpec((blk, F * C), lambda i: (i, 0)),
        out_shape=jax.ShapeDtypeStruct((B, F * C), x.dtype),
    )(x2, bins)
    return out.reshape(B, F, C)


# speedup vs baseline: 1.7080x; 1.1060x over previous
import jax
import jax.numpy as jnp
from jax.experimental import pallas as pl
from jax.experimental.pallas import tpu as pltpu

ALPHA = -50.0


def _body(x_ref, bins_ref, o_ref):
    v = x_ref[...]
    b = [bins_ref[k] for k in range(4)]
    c1 = v < b[1]
    c2 = v < b[2]
    lo = jnp.where(c1, b[0], jnp.where(c2, b[1], b[2]))
    hi = jnp.where(c1, b[1], jnp.where(c2, b[2], b[3]))
    z = ALPHA * (lo + hi - (v + v))
    w = 1.0 / (1.0 + jnp.exp(-z))
    o_ref[...] = lo + (hi - lo) * w


def kernel(x, bins):
    B, F, C = x.shape
    x2 = x.reshape(B, F * C)
    blk = 128
    grid = (B // blk,)
    out = pl.pallas_call(
        _body,
        grid=grid,
        in_specs=[
            pl.BlockSpec((blk, F * C), lambda i: (i, 0)),
            pl.BlockSpec(memory_space=pltpu.SMEM),
        ],
        out_specs=pl.BlockSpec((blk, F * C), lambda i: (i, 0)),
        out_shape=jax.ShapeDtypeStruct((B, F * C), x.dtype),
    )(x2, bins)
    return out.reshape(B, F, C)


# speedup vs baseline: 1.7584x; 1.0295x over previous
import jax
import jax.numpy as jnp
from jax.experimental import pallas as pl
from jax.experimental.pallas import tpu as pltpu

ALPHA = -50.0


def _body(x_ref, bins_ref, o_ref):
    v = x_ref[...]
    b = [bins_ref[k] for k in range(4)]
    c1 = v < b[1]
    c2 = v < b[2]
    lo = jnp.where(c1, b[0], jnp.where(c2, b[1], b[2]))
    hi = jnp.where(c1, b[1], jnp.where(c2, b[2], b[3]))
    z = ALPHA * (lo + hi - (v + v))
    w = 1.0 / (1.0 + jnp.exp(-z))
    o_ref[...] = lo + (hi - lo) * w


def kernel(x, bins):
    B, F, C = x.shape
    x2 = x.reshape(B, F * C)
    blk = 512
    grid = (B // blk,)
    out = pl.pallas_call(
        _body,
        grid=grid,
        in_specs=[
            pl.BlockSpec((blk, F * C), lambda i: (i, 0)),
            pl.BlockSpec(memory_space=pltpu.SMEM),
        ],
        out_specs=pl.BlockSpec((blk, F * C), lambda i: (i, 0)),
        out_shape=jax.ShapeDtypeStruct((B, F * C), x.dtype),
    )(x2, bins)
    return out.reshape(B, F, C)


# speedup vs baseline: 2.6748x; 1.5211x over previous
import jax
import jax.numpy as jnp
from jax.experimental import pallas as pl
from jax.experimental.pallas import tpu as pltpu

ALPHA = -50.0


def _body(x_ref, bins_ref, o_ref):
    v = x_ref[...]
    b = [bins_ref[k] for k in range(4)]
    c1 = v < b[1]
    c2 = v < b[2]
    lo = jnp.where(c1, b[0], jnp.where(c2, b[1], b[2]))
    hi = jnp.where(c1, b[1], jnp.where(c2, b[2], b[3]))
    z = ALPHA * (lo + hi - (v + v))
    w = 1.0 / (1.0 + jnp.exp(-z))
    o_ref[...] = lo + (hi - lo) * w


def kernel(x, bins):
    B, F, C = x.shape
    blk = 256
    grid = (B // blk,)
    out = pl.pallas_call(
        _body,
        grid=grid,
        in_specs=[
            pl.BlockSpec((blk, F, C), lambda i: (i, 0, 0)),
            pl.BlockSpec(memory_space=pltpu.SMEM),
        ],
        out_specs=pl.BlockSpec((blk, F, C), lambda i: (i, 0, 0)),
        out_shape=jax.ShapeDtypeStruct((B, F, C), x.dtype),
    )(x, bins)
    return out


# speedup vs baseline: 9.6393x; 3.6037x over previous
import jax
import jax.numpy as jnp
from jax.experimental import pallas as pl
from jax.experimental.pallas import tpu as pltpu

ALPHA = -50.0


def _body(x_ref, bins_ref, o_ref):
    v = x_ref[...]
    b = [bins_ref[k] for k in range(4)]
    c1 = v < b[1]
    c2 = v < b[2]
    lo = jnp.where(c1, b[0], jnp.where(c2, b[1], b[2]))
    hi = jnp.where(c1, b[1], jnp.where(c2, b[2], b[3]))
    z = ALPHA * (lo + hi - (v + v))
    w = 1.0 / (1.0 + jnp.exp(-z))
    o_ref[...] = lo + (hi - lo) * w


def kernel(x, bins):
    B, F, C = x.shape
    xt = jnp.transpose(x, (1, 0, 2))
    blk = 512
    grid = (B // blk,)
    out = pl.pallas_call(
        _body,
        grid=grid,
        in_specs=[
            pl.BlockSpec((F, blk, C), lambda i: (0, i, 0)),
            pl.BlockSpec(memory_space=pltpu.SMEM),
        ],
        out_specs=pl.BlockSpec((F, blk, C), lambda i: (0, i, 0)),
        out_shape=jax.ShapeDtypeStruct((F, B, C), x.dtype),
    )(xt, bins)
    return jnp.transpose(out, (1, 0, 2))
